# Initial kernel scaffold; baseline (speedup 1.0000x reference)
import jax
import jax.numpy as jnp
from jax.experimental import pallas as pl


def kernel(x, edge_index, edge_attr, W1, b1, g1, bt1, W2, b2, g2, bt2, W3, b3):
    raise NotImplementedError("write your pallas kernel here")


# speedup vs baseline: 6.3638x; 6.3638x over previous
import functools

import jax
import jax.numpy as jnp
from jax import lax
from jax.experimental import pallas as pl
from jax.experimental.pallas import tpu as pltpu
from jax.experimental.pallas import tpu_sc as plsc

_CHUNK = 128
_NW = 32
_PIECE = 400
_PAD = 16


@functools.lru_cache(maxsize=None)
def _make_sc_segsum(N, E, D):
    epw = E // _NW
    nfull = epw // _CHUNK
    tail = epw - nfull * _CHUNK
    nrow = -(-epw // 128)
    assert nfull % 3 == 0 and tail < 128
    ntrip = nfull // 3
    npiece = N // _PIECE
    piece_per_tile = -(-npiece // 16)
    mesh = plsc.VectorSubcoreMesh(core_axis_name="c", subcore_axis_name="s")

    @functools.partial(
        pl.kernel,
        mesh=mesh,
        out_type=jax.ShapeDtypeStruct((2, N, D), jnp.float32),
        scratch_types=[
            pltpu.VMEM((_CHUNK, D), jnp.float32),
            pltpu.VMEM((_CHUNK, D), jnp.float32),
            pltpu.VMEM((_CHUNK, D), jnp.float32),
            pltpu.VMEM((128,), jnp.int32),
            pltpu.VMEM((128,), jnp.int32),
            pltpu.VMEM((128,), jnp.int32),
            pltpu.VMEM_SHARED((N + _PAD, D), jnp.float32),
            pltpu.SemaphoreType.DMA,
            pltpu.SemaphoreType.DMA,
            pltpu.SemaphoreType.DMA,
            pltpu.SemaphoreType.DMA,
            pltpu.SemaphoreType.DMA,
            pltpu.SemaphoreType.DMA,
        ],
    )
    def seg(src_hbm, ea_hbm, z_hbm, out_hbm,
            buf0, buf1, buf2, idx0, idx1, idx2, acc_sh,
            gs0, gs1, gs2, ss0, ss1, ss2):
        bufs = (buf0, buf1, buf2)
        idxs = (idx0, idx1, idx2)
        gsems = (gs0, gs1, gs2)
        ssems = (ss0, ss1, ss2)
        c = lax.axis_index("c")
        s = lax.axis_index("s")
        wid = s * 2 + c
        base_e = wid * epw
        base_i = wid * (nrow * 128)
        for i in range(piece_per_tile):
            pid = i * 16 + s

            @pl.when(pid < npiece)
            def _():
                pltpu.sync_copy(z_hbm, acc_sh.at[pl.ds(pid * _PIECE, _PIECE)])

        plsc.subcore_barrier()

        def g_desc(chunk, b):
            rows = pltpu.make_async_copy(
                ea_hbm.at[pl.ds(base_e + chunk * _CHUNK, _CHUNK)],
                bufs[b], gsems[b])
            idx = pltpu.make_async_copy(
                src_hbm.at[pl.ds(base_i + chunk * 128, 128)],
                idxs[b], gsems[b])
            return rows, idx

        def gather(chunk, b):
            for d in g_desc(chunk, b):
                d.start()

        def wait_gather(chunk, b):
            for d in g_desc(chunk, b):
                d.wait()

        def s_desc(b):
            return pltpu.make_async_copy(bufs[b], acc_sh.at[idxs[b]],
                                         ssems[b])

        def step(ch, b):
            nb = (b + 1) % 3

            @pl.when(ch >= 2)
            def _():
                s_desc(nb).wait()

            @pl.when(ch + 1 < nfull)
            def _():
                gather(ch + 1, nb)

            wait_gather(ch, b)
            pltpu.async_copy(bufs[b], acc_sh.at[idxs[b]], ssems[b], add=True)

        gather(0, 0)

        def body(g, carry):
            c0 = 3 * g
            step(c0, 0)
            step(c0 + 1, 1)
            step(c0 + 2, 2)
            return carry

        lax.fori_loop(0, ntrip, body, 0)
        s_desc((nfull - 2) % 3).wait()
        s_desc((nfull - 1) % 3).wait()
        if tail:
            pltpu.sync_copy(
                ea_hbm.at[pl.ds(base_e + nfull * _CHUNK, tail)],
                buf0.at[pl.ds(0, tail)],
            )
            pltpu.sync_copy(
                src_hbm.at[pl.ds(base_i + (nrow - 1) * 128, 128)], idx0)
            pltpu.sync_copy(buf0, acc_sh.at[idx0], add=True)
        plsc.subcore_barrier()
        for i in range(piece_per_tile):
            pid = i * 16 + s

            @pl.when(pid < npiece)
            def _():
                pltpu.sync_copy(
                    acc_sh.at[pl.ds(pid * _PIECE, _PIECE)],
                    out_hbm.at[c, pl.ds(pid * _PIECE, _PIECE)],
                )

    return seg


def _silu(t):
    return t * jax.nn.sigmoid(t)


def _xw_body(x_ref, w1a, b1r, o_ref):
    o_ref[...] = jnp.dot(x_ref[...], w1a[...].T,
                         preferred_element_type=jnp.float32,
                         precision=lax.Precision.HIGHEST) + b1r[...]


def _mlp_body(xw_ref, p_ref, w1b, g1r, bt1r, w2, b2r, g2r, bt2r,
              w3, b3r, o_ref):
    hp = jnp.float32
    agg = p_ref[0] + p_ref[1]
    t = xw_ref[...] + jnp.dot(agg, w1b[...].T, preferred_element_type=hp,
                              precision=lax.Precision.HIGHEST)
    t = _silu(t)
    mu = jnp.mean(t, axis=0, keepdims=True)
    d = t - mu
    var = jnp.mean(d * d, axis=0, keepdims=True)
    t = d * lax.rsqrt(var + 1e-5) * g1r[...] + bt1r[...]
    t = jnp.dot(t, w2[...].T, preferred_element_type=hp,
                precision=lax.Precision.HIGHEST)
    t = _silu(t + b2r[...])
    mu = jnp.mean(t, axis=0, keepdims=True)
    d = t - mu
    var = jnp.mean(d * d, axis=0, keepdims=True)
    t = d * lax.rsqrt(var + 1e-5) * g2r[...] + bt2r[...]
    o_ref[...] = jnp.dot(t, w3[...].T, preferred_element_type=hp,
                         precision=lax.Precision.HIGHEST) + b3r[...]


def kernel(x, edge_index, edge_attr, W1, b1, g1, bt1, W2, b2, g2, bt2, W3, b3):
    N, D = x.shape
    E = edge_attr.shape[0]
    epw = E // _NW
    nrow = -(-epw // 128)
    pad_len = nrow * 128 - epw
    srcw = edge_index[0].reshape(_NW, epw)
    pad = jnp.broadcast_to(
        (jnp.arange(pad_len, dtype=jnp.int32) % _PAD) + N, (_NW, pad_len))
    src_flat = jnp.concatenate([srcw, pad], axis=1).reshape(_NW * nrow * 128)
    z = jnp.zeros((_PIECE, D), jnp.float32)
    parts = _make_sc_segsum(N, E, D)(src_flat, edge_attr, z)
    xw = pl.pallas_call(
        _xw_body,
        out_shape=jax.ShapeDtypeStruct((N, D), jnp.float32),
    )(x, W1[:, :D], b1[None])
    out = pl.pallas_call(
        _mlp_body,
        out_shape=jax.ShapeDtypeStruct((N, D), jnp.float32),
    )(xw, parts, W1[:, D:], g1[None], bt1[None],
      W2, b2[None], g2[None], bt2[None], W3, b3[None])
    return out


# speedup vs baseline: 6.6152x; 1.0395x over previous
import functools

import jax
import jax.numpy as jnp
from jax import lax
from jax.experimental import pallas as pl
from jax.experimental.pallas import tpu as pltpu
from jax.experimental.pallas import tpu_sc as plsc

_CHUNK = 128
_NW = 32


@functools.lru_cache(maxsize=None)
def _make_sc_segsum(N, E, D):
    nchunk = E // _CHUNK
    ntrip = nchunk // (3 * _NW)
    nfull = 3 * ntrip
    epw = nfull * _CHUNK
    nleft = nchunk - _NW * nfull
    left_base = _NW * epw
    assert E == _NW * epw + nleft * _CHUNK and nleft <= _NW
    npiece = -(-N // _CHUNK)
    lastp = N - (npiece - 1) * _CHUNK
    piece_per_tile = -(-npiece // 16)
    mesh = plsc.VectorSubcoreMesh(core_axis_name="c", subcore_axis_name="s")

    @functools.partial(
        pl.kernel,
        mesh=mesh,
        out_type=jax.ShapeDtypeStruct((2, N, D), jnp.float32),
        scratch_types=[
            pltpu.VMEM((_CHUNK, D), jnp.float32),
            pltpu.VMEM((_CHUNK, D), jnp.float32),
            pltpu.VMEM((_CHUNK, D), jnp.float32),
            pltpu.VMEM((128,), jnp.int32),
            pltpu.VMEM((128,), jnp.int32),
            pltpu.VMEM((128,), jnp.int32),
            pltpu.VMEM_SHARED((N, D), jnp.float32),
            pltpu.SemaphoreType.DMA,
            pltpu.SemaphoreType.DMA,
            pltpu.SemaphoreType.DMA,
            pltpu.SemaphoreType.DMA,
            pltpu.SemaphoreType.DMA,
            pltpu.SemaphoreType.DMA,
        ],
    )
    def seg(src_hbm, ea_hbm, z_hbm, out_hbm,
            buf0, buf1, buf2, idx0, idx1, idx2, acc_sh,
            gs0, gs1, gs2, ss0, ss1, ss2):
        bufs = (buf0, buf1, buf2)
        idxs = (idx0, idx1, idx2)
        gsems = (gs0, gs1, gs2)
        ssems = (ss0, ss1, ss2)
        c = lax.axis_index("c")
        s = lax.axis_index("s")
        wid = s * 2 + c
        base_e = wid * epw

        pltpu.sync_copy(z_hbm, buf2)
        for i in range(piece_per_tile):
            pid = i * 16 + s

            @pl.when(pid < npiece - 1)
            def _():
                pltpu.sync_copy(buf2,
                                acc_sh.at[pl.ds(pid * _CHUNK, _CHUNK)])

            @pl.when(pid == npiece - 1)
            def _():
                pltpu.sync_copy(buf2.at[pl.ds(0, lastp)],
                                acc_sh.at[pl.ds(pid * _CHUNK, lastp)])

        plsc.subcore_barrier()

        def g_desc(chunk, b):
            rows = pltpu.make_async_copy(
                ea_hbm.at[pl.ds(base_e + chunk * _CHUNK, _CHUNK)],
                bufs[b], gsems[b])
            idx = pltpu.make_async_copy(
                src_hbm.at[pl.ds(base_e + chunk * _CHUNK, 128)],
                idxs[b], gsems[b])
            return rows, idx

        def gather(chunk, b):
            for d in g_desc(chunk, b):
                d.start()

        def wait_gather(chunk, b):
            for d in g_desc(chunk, b):
                d.wait()

        def s_desc(b):
            return pltpu.make_async_copy(bufs[b], acc_sh.at[idxs[b]],
                                         ssems[b])

        def step(ch, b):
            nb = (b + 1) % 3

            @pl.when(ch >= 2)
            def _():
                s_desc(nb).wait()

            @pl.when(ch + 1 < nfull)
            def _():
                gather(ch + 1, nb)

            wait_gather(ch, b)
            pltpu.async_copy(bufs[b], acc_sh.at[idxs[b]], ssems[b], add=True)

        gather(0, 0)

        def body(g, carry):
            c0 = 3 * g
            step(c0, 0)
            step(c0 + 1, 1)
            step(c0 + 2, 2)
            return carry

        lax.fori_loop(0, ntrip, body, 0)
        s_desc((nfull - 2) % 3).wait()
        s_desc((nfull - 1) % 3).wait()
        if nleft:
            @pl.when(wid < nleft)
            def _():
                off = left_base + wid * _CHUNK
                pltpu.sync_copy(ea_hbm.at[pl.ds(off, _CHUNK)], buf0)
                pltpu.sync_copy(src_hbm.at[pl.ds(off, 128)], idx0)
                pltpu.sync_copy(buf0, acc_sh.at[idx0], add=True)

        plsc.subcore_barrier()

        def w_descs(i):
            pid = i * 16 + s
            b = i % 3
            full = pltpu.make_async_copy(
                bufs[b], out_hbm.at[c, pl.ds(pid * _CHUNK, _CHUNK)],
                ssems[b])
            last = pltpu.make_async_copy(
                bufs[b].at[pl.ds(0, lastp)],
                out_hbm.at[c, pl.ds(pid * _CHUNK, lastp)], ssems[b])
            return pid, b, full, last

        def w_issue(i):
            pid, b, full, last = w_descs(i)

            @pl.when(pid < npiece - 1)
            def _():
                pltpu.sync_copy(acc_sh.at[pl.ds(pid * _CHUNK, _CHUNK)],
                                bufs[b])
                full.start()

            @pl.when(pid == npiece - 1)
            def _():
                pltpu.sync_copy(acc_sh.at[pl.ds(pid * _CHUNK, lastp)],
                                bufs[b].at[pl.ds(0, lastp)])
                last.start()

        def w_wait(i):
            pid, b, full, last = w_descs(i)

            @pl.when(pid < npiece - 1)
            def _():
                full.wait()

            @pl.when(pid == npiece - 1)
            def _():
                last.wait()

        for i in range(piece_per_tile):
            if i >= 3:
                w_wait(i - 3)
            w_issue(i)
        for i in range(max(0, piece_per_tile - 3), piece_per_tile):
            w_wait(i)

    return seg


def _silu(t):
    return t * jax.nn.sigmoid(t)


def _xw_body(x_ref, w1a, b1r, o_ref):
    o_ref[...] = jnp.dot(x_ref[...], w1a[...].T,
                         preferred_element_type=jnp.float32,
                         precision=lax.Precision.HIGHEST) + b1r[...]


def _mlp_body(xw_ref, p_ref, w1b, g1r, bt1r, w2, b2r, g2r, bt2r,
              w3, b3r, o_ref):
    hp = jnp.float32
    agg = p_ref[0] + p_ref[1]
    t = xw_ref[...] + jnp.dot(agg, w1b[...].T, preferred_element_type=hp,
                              precision=lax.Precision.HIGHEST)
    t = _silu(t)
    mu = jnp.mean(t, axis=0, keepdims=True)
    d = t - mu
    var = jnp.mean(d * d, axis=0, keepdims=True)
    t = d * lax.rsqrt(var + 1e-5) * g1r[...] + bt1r[...]
    t = jnp.dot(t, w2[...].T, preferred_element_type=hp,
                precision=lax.Precision.HIGHEST)
    t = _silu(t + b2r[...])
    mu = jnp.mean(t, axis=0, keepdims=True)
    d = t - mu
    var = jnp.mean(d * d, axis=0, keepdims=True)
    t = d * lax.rsqrt(var + 1e-5) * g2r[...] + bt2r[...]
    o_ref[...] = jnp.dot(t, w3[...].T, preferred_element_type=hp,
                         precision=lax.Precision.HIGHEST) + b3r[...]


def kernel(x, edge_index, edge_attr, W1, b1, g1, bt1, W2, b2, g2, bt2, W3, b3):
    N, D = x.shape
    E = edge_attr.shape[0]
    src = edge_index[0]
    z = jnp.zeros((_CHUNK, D), jnp.float32)
    parts = _make_sc_segsum(N, E, D)(src, edge_attr, z)
    xw = pl.pallas_call(
        _xw_body,
        out_shape=jax.ShapeDtypeStruct((N, D), jnp.float32),
    )(x, W1[:, :D], b1[None])
    out = pl.pallas_call(
        _mlp_body,
        out_shape=jax.ShapeDtypeStruct((N, D), jnp.float32),
    )(xw, parts, W1[:, D:], g1[None], bt1[None],
      W2, b2[None], g2[None], bt2[None], W3, b3[None])
    return out


# speedup vs baseline: 7.9844x; 1.2070x over previous
import functools

import jax
import jax.numpy as jnp
from jax import lax
from jax.experimental import pallas as pl
from jax.experimental.pallas import tpu as pltpu
from jax.experimental.pallas import tpu_sc as plsc

_CHUNK = 128
_NW = 32


@functools.lru_cache(maxsize=None)
def _make_sc_segsum(N, E, D):
    nchunk = E // _CHUNK
    ntrip = nchunk // (3 * _NW)
    nfull = 3 * ntrip
    epw = nfull * _CHUNK
    nleft = nchunk - _NW * nfull
    left_base = _NW * epw
    assert E == _NW * epw + nleft * _CHUNK and nleft <= _NW
    npiece = -(-N // _CHUNK)
    lastp = N - (npiece - 1) * _CHUNK
    piece_per_tile = -(-npiece // 16)
    mesh = plsc.VectorSubcoreMesh(core_axis_name="c", subcore_axis_name="s")

    @functools.partial(
        pl.kernel,
        mesh=mesh,
        out_type=jax.ShapeDtypeStruct((2, N, D), jnp.float32),
        scratch_types=[
            pltpu.VMEM((_CHUNK, D), jnp.float32),
            pltpu.VMEM((_CHUNK, D), jnp.float32),
            pltpu.VMEM((_CHUNK, D), jnp.float32),
            pltpu.VMEM((128,), jnp.int32),
            pltpu.VMEM((128,), jnp.int32),
            pltpu.VMEM((128,), jnp.int32),
            pltpu.VMEM_SHARED((N, D), jnp.float32),
            pltpu.SemaphoreType.DMA,
            pltpu.SemaphoreType.DMA,
            pltpu.SemaphoreType.DMA,
            pltpu.SemaphoreType.DMA,
            pltpu.SemaphoreType.DMA,
            pltpu.SemaphoreType.DMA,
        ],
    )
    def seg(src_hbm, ea_hbm, z_hbm, out_hbm,
            buf0, buf1, buf2, idx0, idx1, idx2, acc_sh,
            gs0, gs1, gs2, ss0, ss1, ss2):
        bufs = (buf0, buf1, buf2)
        idxs = (idx0, idx1, idx2)
        gsems = (gs0, gs1, gs2)
        ssems = (ss0, ss1, ss2)
        c = lax.axis_index("c")
        s = lax.axis_index("s")
        wid = s * 2 + c
        base_e = wid * epw

        pltpu.sync_copy(z_hbm, buf2)
        for i in range(piece_per_tile):
            pid = i * 16 + s

            @pl.when(pid < npiece - 1)
            def _():
                pltpu.sync_copy(buf2,
                                acc_sh.at[pl.ds(pid * _CHUNK, _CHUNK)])

            @pl.when(pid == npiece - 1)
            def _():
                pltpu.sync_copy(buf2.at[pl.ds(0, lastp)],
                                acc_sh.at[pl.ds(pid * _CHUNK, lastp)])

        plsc.subcore_barrier()

        def g_desc(chunk, b):
            rows = pltpu.make_async_copy(
                ea_hbm.at[pl.ds(base_e + chunk * _CHUNK, _CHUNK)],
                bufs[b], gsems[b])
            idx = pltpu.make_async_copy(
                src_hbm.at[0, pl.ds(base_e + chunk * _CHUNK, 128)],
                idxs[b], gsems[b])
            return rows, idx

        def gather(chunk, b):
            for d in g_desc(chunk, b):
                d.start()

        def wait_gather(chunk, b):
            for d in g_desc(chunk, b):
                d.wait()

        def s_desc(b):
            return pltpu.make_async_copy(bufs[b], acc_sh.at[idxs[b]],
                                         ssems[b])

        def step(ch, b):
            nb = (b + 1) % 3

            @pl.when(ch >= 2)
            def _():
                s_desc(nb).wait()

            @pl.when(ch + 1 < nfull)
            def _():
                gather(ch + 1, nb)

            wait_gather(ch, b)
            pltpu.async_copy(bufs[b], acc_sh.at[idxs[b]], ssems[b], add=True)

        gather(0, 0)

        def body(g, carry):
            c0 = 3 * g
            step(c0, 0)
            step(c0 + 1, 1)
            step(c0 + 2, 2)
            return carry

        lax.fori_loop(0, ntrip, body, 0)
        s_desc((nfull - 2) % 3).wait()
        s_desc((nfull - 1) % 3).wait()
        if nleft:
            @pl.when(wid < nleft)
            def _():
                off = left_base + wid * _CHUNK
                pltpu.sync_copy(ea_hbm.at[pl.ds(off, _CHUNK)], buf0)
                pltpu.sync_copy(src_hbm.at[0, pl.ds(off, 128)], idx0)
                pltpu.sync_copy(buf0, acc_sh.at[idx0], add=True)

        plsc.subcore_barrier()

        def w_descs(i):
            pid = i * 16 + s
            b = i % 3
            full = pltpu.make_async_copy(
                bufs[b], out_hbm.at[c, pl.ds(pid * _CHUNK, _CHUNK)],
                ssems[b])
            last = pltpu.make_async_copy(
                bufs[b].at[pl.ds(0, lastp)],
                out_hbm.at[c, pl.ds(pid * _CHUNK, lastp)], ssems[b])
            return pid, b, full, last

        def w_issue(i):
            pid, b, full, last = w_descs(i)

            @pl.when(pid < npiece - 1)
            def _():
                pltpu.sync_copy(acc_sh.at[pl.ds(pid * _CHUNK, _CHUNK)],
                                bufs[b])
                full.start()

            @pl.when(pid == npiece - 1)
            def _():
                pltpu.sync_copy(acc_sh.at[pl.ds(pid * _CHUNK, lastp)],
                                bufs[b].at[pl.ds(0, lastp)])
                last.start()

        def w_wait(i):
            pid, b, full, last = w_descs(i)

            @pl.when(pid < npiece - 1)
            def _():
                full.wait()

            @pl.when(pid == npiece - 1)
            def _():
                last.wait()

        for i in range(piece_per_tile):
            if i >= 3:
                w_wait(i - 3)
            w_issue(i)
        for i in range(max(0, piece_per_tile - 3), piece_per_tile):
            w_wait(i)

    return seg


def _silu(t):
    return t * jax.nn.sigmoid(t)


def _xw_body(x_ref, w1a, b1r, o_ref):
    o_ref[...] = jnp.dot(x_ref[...], w1a[...].T,
                         preferred_element_type=jnp.float32,
                         precision=lax.Precision.DEFAULT) + b1r[...]


def _mlp_body(xw_ref, p_ref, w1b, g1r, bt1r, w2, b2r, g2r, bt2r,
              w3, b3r, o_ref):
    hp = jnp.float32
    agg = p_ref[0] + p_ref[1]
    t = xw_ref[...] + jnp.dot(agg, w1b[...].T, preferred_element_type=hp,
                              precision=lax.Precision.DEFAULT)
    t = _silu(t)
    mu = jnp.mean(t, axis=0, keepdims=True)
    d = t - mu
    var = jnp.mean(d * d, axis=0, keepdims=True)
    t = d * lax.rsqrt(var + 1e-5) * g1r[...] + bt1r[...]
    t = jnp.dot(t, w2[...].T, preferred_element_type=hp,
                precision=lax.Precision.DEFAULT)
    t = _silu(t + b2r[...])
    mu = jnp.mean(t, axis=0, keepdims=True)
    d = t - mu
    var = jnp.mean(d * d, axis=0, keepdims=True)
    t = d * lax.rsqrt(var + 1e-5) * g2r[...] + bt2r[...]
    o_ref[...] = jnp.dot(t, w3[...].T, preferred_element_type=hp,
                         precision=lax.Precision.DEFAULT) + b3r[...]


def kernel(x, edge_index, edge_attr, W1, b1, g1, bt1, W2, b2, g2, bt2, W3, b3):
    N, D = x.shape
    E = edge_attr.shape[0]
    z = jnp.zeros((_CHUNK, D), jnp.float32)
    parts = _make_sc_segsum(N, E, D)(edge_index, edge_attr, z)
    xw = pl.pallas_call(
        _xw_body,
        out_shape=jax.ShapeDtypeStruct((N, D), jnp.float32),
    )(x, W1[:, :D], b1[None])
    out = pl.pallas_call(
        _mlp_body,
        out_shape=jax.ShapeDtypeStruct((N, D), jnp.float32),
    )(xw, parts, W1[:, D:], g1[None], bt1[None],
      W2, b2[None], g2[None], bt2[None], W3, b3[None])
    return out
